# baseline (speedup 1.0000x reference)
import functools

import jax
import jax.numpy as jnp
from jax.experimental import pallas as pl
from jax.experimental.pallas import tpu as pltpu

KNN = 16
ROWS = 512


def _dot(a, b, dims):
    return jax.lax.dot_general(a, b, (dims, ((), ())),
                               preferred_element_type=jnp.float32)


def _layer_kernel(h_ref, wt_ref, bt_ref, wc_ref, wa_ref, ba_ref,
                  wm_ref, bm_ref, wl_ref, bl_ref,
                  wm1_ref, wl1_ref, wl2_ref, out_ref,
                  ht_ref, c_ref, a_ref, m_ref, l_ref, *, act):
    t = pl.program_id(1)
    n = h_ref.shape[1]
    rows = ROWS

    @pl.when(t == 0)
    def _prep():
        h = h_ref[0]
        ht = _dot(h, wt_ref[...], ((1,), (0,))) + bt_ref[...]
        if act:
            ht = jnp.maximum(ht, 0.0)
        d2 = jnp.sum(ht * ht, axis=1, keepdims=True)
        ht_ref[...] = jnp.concatenate([ht, d2], axis=1)
        c = _dot(ht, wc_ref[...], ((1,), (0,)))
        c_ref[...] = jnp.concatenate(
            [c, jnp.ones((n, 1), jnp.float32),
             jnp.zeros((n, 3), jnp.float32)], axis=1)
        a_ref[...] = _dot(ht, wa_ref[...], ((1,), (0,))) + ba_ref[...]
        m_ref[...] = _dot(ht, wm_ref[...], ((1,), (0,))) + bm_ref[...]
        l_ref[...] = _dot(ht, wl_ref[...], ((1,), (0,))) + bl_ref[...]

    @pl.when(t > 0)
    def _edge():
        base = (t - 1) * rows
        ht_aug = ht_ref[...]
        xt = ht_ref[pl.ds(base, rows), 0:24]

        xt_aug = jnp.concatenate(
            [xt * -2.0, jnp.ones((rows, 1), jnp.float32)], axis=1)
        dist = _dot(xt_aug, ht_aug, ((1,), (1,)))
        dist = dist + jnp.sum(xt * xt, axis=1, keepdims=True)
        row_i = jax.lax.broadcasted_iota(jnp.int32, (rows, n), 0) + base
        col_i = jax.lax.broadcasted_iota(jnp.int32, (rows, n), 1)
        dist = jnp.where(row_i == col_i, jnp.inf, dist)

        c_all = c_ref[...]
        cgs = []
        for s in range(KNN):
            mn = jnp.min(dist, axis=1, keepdims=True)
            eq = dist == mn
            ohf = jnp.where(eq, 1.0, 0.0)
            g = _dot(ohf, c_all, ((1,), (0,)))
            cgs.append(g[:, 0:12] * (1.0 / g[:, 12:13]))
            dist = jnp.where(eq, jnp.inf, dist)
        cg = jnp.concatenate(cgs, axis=0)

        a_t = a_ref[pl.ds(base, rows), :]
        m_t = m_ref[pl.ds(base, rows), :]
        l_t = l_ref[pl.ds(base, rows), :]
        a_rep = jnp.concatenate([a_t] * KNN, axis=0)
        m_rep = jnp.concatenate([m_t] * KNN, axis=0)
        l_rep = jnp.concatenate([l_t] * KNN, axis=0)

        r = jnp.maximum(a_rep + cg, 0.0)
        m = jnp.maximum(_dot(r, wm1_ref[...], ((1,), (0,))) + m_rep, 0.0)
        l = (_dot(m, wl1_ref[...], ((1,), (0,)))
             + _dot(r, wl2_ref[...], ((1,), (0,))) + l_rep)

        mr = r[0:rows]
        mm = m[0:rows]
        ml = l[0:rows]
        for k in range(1, KNN):
            sl = slice(k * rows, (k + 1) * rows)
            mr = jnp.maximum(mr, r[sl])
            mm = jnp.maximum(mm, m[sl])
            ml = jnp.maximum(ml, l[sl])

        out_ref[0] = jnp.concatenate([ml, mm, mr, xt], axis=1)


def _layer(h, p, i):
    bsz, n, in_ch = h.shape
    wt = p[f"trans{i}_W"]
    bt = p[f"trans{i}_b"][None, :]
    wf = p[f"conv{i}_first_W"]
    bf = p[f"conv{i}_first_b"][None, :]
    if i == 0:
        wc = wf
        wa = -wf
    else:
        wa = wf[:24] - wf[48:]
        wc = wf[24:48] + wf[48:]
    wm = p[f"conv{i}_mid0_W"]
    bm = p[f"conv{i}_mid0_b"][None, :]
    wm1, wm2 = wm[:12], wm[12:]
    wl = p[f"conv{i}_last_W"]
    bl = p[f"conv{i}_last_b"][None, :]
    wl1, wl2, wl3 = wl[:12], wl[12:24], wl[24:]

    def wspec(w):
        return pl.BlockSpec(w.shape, lambda b_, t_: (0,) * w.ndim)

    nt = n // ROWS

    out = pl.pallas_call(
        functools.partial(_layer_kernel, act=(i != 0)),
        grid=(bsz, nt + 1),
        in_specs=[pl.BlockSpec((1, n, in_ch), lambda b_, t_: (b_, 0, 0))]
        + [wspec(w) for w in (wt, bt, wc, wa, bf, wm2, bm, wl3, bl,
                              wm1, wl1, wl2)],
        out_specs=pl.BlockSpec(
            (1, ROWS, 60),
            lambda b_, t_: (b_, jnp.maximum(t_ - 1, 0), 0)),
        out_shape=jax.ShapeDtypeStruct((bsz, n, 60), jnp.float32),
        scratch_shapes=[
            pltpu.VMEM((n, 25), jnp.float32),
            pltpu.VMEM((n, 16), jnp.float32),
            pltpu.VMEM((n, 12), jnp.float32),
            pltpu.VMEM((n, 12), jnp.float32),
            pltpu.VMEM((n, 12), jnp.float32),
        ],
    )(h, wt, bt, wc, wa, bf, wm2, bm, wl3, bl, wm1, wl1, wl2)
    return out


def kernel(x, params):
    h = x
    for i in range(4):
        h = _layer(h, params, i)
    return h


# speedup vs baseline: 1.0677x; 1.0677x over previous
import functools

import jax
import jax.numpy as jnp
from jax.experimental import pallas as pl
from jax.experimental.pallas import tpu as pltpu

KNN = 16
ROWS = 256


def _dot(a, b, dims):
    return jax.lax.dot_general(a, b, (dims, ((), ())),
                               preferred_element_type=jnp.float32)


def _layer_kernel(h_ref, wt_ref, bt_ref, wc_ref, wa_ref, ba_ref,
                  wm_ref, bm_ref, wl_ref, bl_ref,
                  wm1_ref, wl1_ref, wl2_ref, out_ref,
                  ht_ref, c_ref, a_ref, m_ref, l_ref, *, act):
    t = pl.program_id(1)
    n = h_ref.shape[1]
    rows = ROWS

    @pl.when(t == 0)
    def _prep():
        h = h_ref[0]
        ht = _dot(h, wt_ref[...], ((1,), (0,))) + bt_ref[...]
        if act:
            ht = jnp.maximum(ht, 0.0)
        d2 = jnp.sum(ht * ht, axis=1, keepdims=True)
        ht_ref[...] = jnp.concatenate([ht, d2], axis=1)
        c = _dot(ht, wc_ref[...], ((1,), (0,)))
        c_ref[...] = jnp.concatenate(
            [c, jnp.ones((n, 1), jnp.float32),
             jnp.zeros((n, 3), jnp.float32)], axis=1)
        a_ref[...] = _dot(ht, wa_ref[...], ((1,), (0,))) + ba_ref[...]
        m_ref[...] = _dot(ht, wm_ref[...], ((1,), (0,))) + bm_ref[...]
        l_ref[...] = _dot(ht, wl_ref[...], ((1,), (0,))) + bl_ref[...]

    @pl.when(t > 0)
    def _edge():
        base = (t - 1) * rows
        ht_aug = ht_ref[...]
        xt = ht_ref[pl.ds(base, rows), 0:24]

        xt_aug = jnp.concatenate(
            [xt * -2.0, jnp.ones((rows, 1), jnp.float32)], axis=1)
        dist = _dot(xt_aug, ht_aug, ((1,), (1,)))
        dist = dist + jnp.sum(xt * xt, axis=1, keepdims=True)
        row_i = jax.lax.broadcasted_iota(jnp.int32, (rows, n), 0) + base
        col_i = jax.lax.broadcasted_iota(jnp.int32, (rows, n), 1)
        dist = jnp.where(row_i == col_i, jnp.inf, dist)

        c_all = c_ref[...]
        cgs = []
        for s in range(KNN):
            mn = jnp.min(dist, axis=1, keepdims=True)
            eq = dist == mn
            ohf = jnp.where(eq, 1.0, 0.0)
            g = _dot(ohf, c_all, ((1,), (0,)))
            cgs.append(g[:, 0:12] * (1.0 / g[:, 12:13]))
            dist = jnp.where(eq, jnp.inf, dist)
        cg = jnp.concatenate(cgs, axis=0)

        a_t = a_ref[pl.ds(base, rows), :]
        m_t = m_ref[pl.ds(base, rows), :]
        l_t = l_ref[pl.ds(base, rows), :]
        a_rep = jnp.concatenate([a_t] * KNN, axis=0)
        m_rep = jnp.concatenate([m_t] * KNN, axis=0)
        l_rep = jnp.concatenate([l_t] * KNN, axis=0)

        r = jnp.maximum(a_rep + cg, 0.0)
        m = jnp.maximum(_dot(r, wm1_ref[...], ((1,), (0,))) + m_rep, 0.0)
        l = (_dot(m, wl1_ref[...], ((1,), (0,)))
             + _dot(r, wl2_ref[...], ((1,), (0,))) + l_rep)

        mr = r[0:rows]
        mm = m[0:rows]
        ml = l[0:rows]
        for k in range(1, KNN):
            sl = slice(k * rows, (k + 1) * rows)
            mr = jnp.maximum(mr, r[sl])
            mm = jnp.maximum(mm, m[sl])
            ml = jnp.maximum(ml, l[sl])

        out_ref[0] = jnp.concatenate([ml, mm, mr, xt], axis=1)


def _layer(h, p, i):
    bsz, n, in_ch = h.shape
    wt = p[f"trans{i}_W"]
    bt = p[f"trans{i}_b"][None, :]
    wf = p[f"conv{i}_first_W"]
    bf = p[f"conv{i}_first_b"][None, :]
    if i == 0:
        wc = wf
        wa = -wf
    else:
        wa = wf[:24] - wf[48:]
        wc = wf[24:48] + wf[48:]
    wm = p[f"conv{i}_mid0_W"]
    bm = p[f"conv{i}_mid0_b"][None, :]
    wm1, wm2 = wm[:12], wm[12:]
    wl = p[f"conv{i}_last_W"]
    bl = p[f"conv{i}_last_b"][None, :]
    wl1, wl2, wl3 = wl[:12], wl[12:24], wl[24:]

    def wspec(w):
        return pl.BlockSpec(w.shape, lambda b_, t_: (0,) * w.ndim)

    nt = n // ROWS

    out = pl.pallas_call(
        functools.partial(_layer_kernel, act=(i != 0)),
        grid=(bsz, nt + 1),
        in_specs=[pl.BlockSpec((1, n, in_ch), lambda b_, t_: (b_, 0, 0))]
        + [wspec(w) for w in (wt, bt, wc, wa, bf, wm2, bm, wl3, bl,
                              wm1, wl1, wl2)],
        out_specs=pl.BlockSpec(
            (1, ROWS, 60),
            lambda b_, t_: (b_, jnp.maximum(t_ - 1, 0), 0)),
        out_shape=jax.ShapeDtypeStruct((bsz, n, 60), jnp.float32),
        scratch_shapes=[
            pltpu.VMEM((n, 25), jnp.float32),
            pltpu.VMEM((n, 16), jnp.float32),
            pltpu.VMEM((n, 12), jnp.float32),
            pltpu.VMEM((n, 12), jnp.float32),
            pltpu.VMEM((n, 12), jnp.float32),
        ],
    )(h, wt, bt, wc, wa, bf, wm2, bm, wl3, bl, wm1, wl1, wl2)
    return out


def kernel(x, params):
    h = x
    for i in range(4):
        h = _layer(h, params, i)
    return h


# speedup vs baseline: 1.0701x; 1.0022x over previous
import functools

import jax
import jax.numpy as jnp
from jax.experimental import pallas as pl
from jax.experimental.pallas import tpu as pltpu

KNN = 16
ROWS = 256


def _dot(a, b, dims):
    return jax.lax.dot_general(a, b, (dims, ((), ())),
                               preferred_element_type=jnp.float32)


def _layer_kernel(h_ref, wt_ref, bt_ref, wc_ref, wa_ref, ba_ref,
                  wm_ref, bm_ref, wl_ref, bl_ref,
                  wm1_ref, wl1_ref, wl2_ref, out_ref,
                  ht_ref, c_ref, a_ref, m_ref, l_ref, *, act):
    t = pl.program_id(1)
    n = h_ref.shape[1]
    rows = ROWS

    @pl.when(t == 0)
    def _prep():
        h = h_ref[0]
        ht = _dot(h, wt_ref[...], ((1,), (0,))) + bt_ref[...]
        if act:
            ht = jnp.maximum(ht, 0.0)
        d2 = jnp.sum(ht * ht, axis=1, keepdims=True)
        ht_ref[...] = jnp.concatenate([ht, d2], axis=1)
        c = _dot(ht, wc_ref[...], ((1,), (0,)))
        c_ref[...] = jnp.concatenate(
            [c, jnp.ones((n, 1), jnp.float32),
             jnp.zeros((n, 3), jnp.float32)], axis=1)
        a_ref[...] = _dot(ht, wa_ref[...], ((1,), (0,))) + ba_ref[...]
        m_ref[...] = _dot(ht, wm_ref[...], ((1,), (0,))) + bm_ref[...]
        l_ref[...] = _dot(ht, wl_ref[...], ((1,), (0,))) + bl_ref[...]

    @pl.when(t > 0)
    def _edge():
        base = (t - 1) * rows
        ht_aug = ht_ref[...]
        xt = ht_ref[pl.ds(base, rows), 0:24]

        xt_aug = jnp.concatenate(
            [xt * -2.0, jnp.ones((rows, 1), jnp.float32)], axis=1)
        dist = _dot(xt_aug, ht_aug, ((1,), (1,)))
        dist = dist + jnp.sum(xt * xt, axis=1, keepdims=True)
        row_i = jax.lax.broadcasted_iota(jnp.int32, (rows, n), 0) + base
        col_i = jax.lax.broadcasted_iota(jnp.int32, (rows, n), 1)
        dist = jnp.where(row_i == col_i, jnp.inf, dist)

        c_all = c_ref[...]
        cgs = []
        for s in range(KNN):
            mn = jnp.min(dist, axis=1, keepdims=True)
            eq = dist == mn
            ohf = jnp.where(eq, 1.0, 0.0)
            cgs.append(_dot(ohf, c_all, ((1,), (0,))))
            dist = jnp.where(eq, jnp.inf, dist)
        graw = jnp.concatenate(cgs, axis=0)
        cg = graw[:, 0:12] * (1.0 / graw[:, 12:13])

        a_t = a_ref[pl.ds(base, rows), :]
        m_t = m_ref[pl.ds(base, rows), :]
        l_t = l_ref[pl.ds(base, rows), :]
        a_rep = jnp.concatenate([a_t] * KNN, axis=0)
        m_rep = jnp.concatenate([m_t] * KNN, axis=0)
        l_rep = jnp.concatenate([l_t] * KNN, axis=0)

        r = jnp.maximum(a_rep + cg, 0.0)
        m = jnp.maximum(_dot(r, wm1_ref[...], ((1,), (0,))) + m_rep, 0.0)
        l = (_dot(m, wl1_ref[...], ((1,), (0,)))
             + _dot(r, wl2_ref[...], ((1,), (0,))) + l_rep)

        mr = r[0:rows]
        mm = m[0:rows]
        ml = l[0:rows]
        for k in range(1, KNN):
            sl = slice(k * rows, (k + 1) * rows)
            mr = jnp.maximum(mr, r[sl])
            mm = jnp.maximum(mm, m[sl])
            ml = jnp.maximum(ml, l[sl])

        out_ref[0] = jnp.concatenate([ml, mm, mr, xt], axis=1)


def _layer(h, p, i):
    bsz, n, in_ch = h.shape
    wt = p[f"trans{i}_W"]
    bt = p[f"trans{i}_b"][None, :]
    wf = p[f"conv{i}_first_W"]
    bf = p[f"conv{i}_first_b"][None, :]
    if i == 0:
        wc = wf
        wa = -wf
    else:
        wa = wf[:24] - wf[48:]
        wc = wf[24:48] + wf[48:]
    wm = p[f"conv{i}_mid0_W"]
    bm = p[f"conv{i}_mid0_b"][None, :]
    wm1, wm2 = wm[:12], wm[12:]
    wl = p[f"conv{i}_last_W"]
    bl = p[f"conv{i}_last_b"][None, :]
    wl1, wl2, wl3 = wl[:12], wl[12:24], wl[24:]

    def wspec(w):
        return pl.BlockSpec(w.shape, lambda b_, t_: (0,) * w.ndim)

    nt = n // ROWS

    out = pl.pallas_call(
        functools.partial(_layer_kernel, act=(i != 0)),
        grid=(bsz, nt + 1),
        in_specs=[pl.BlockSpec((1, n, in_ch), lambda b_, t_: (b_, 0, 0))]
        + [wspec(w) for w in (wt, bt, wc, wa, bf, wm2, bm, wl3, bl,
                              wm1, wl1, wl2)],
        out_specs=pl.BlockSpec(
            (1, ROWS, 60),
            lambda b_, t_: (b_, jnp.maximum(t_ - 1, 0), 0)),
        out_shape=jax.ShapeDtypeStruct((bsz, n, 60), jnp.float32),
        scratch_shapes=[
            pltpu.VMEM((n, 25), jnp.float32),
            pltpu.VMEM((n, 16), jnp.float32),
            pltpu.VMEM((n, 12), jnp.float32),
            pltpu.VMEM((n, 12), jnp.float32),
            pltpu.VMEM((n, 12), jnp.float32),
        ],
    )(h, wt, bt, wc, wa, bf, wm2, bm, wl3, bl, wm1, wl1, wl2)
    return out


def kernel(x, params):
    h = x
    for i in range(4):
        h = _layer(h, params, i)
    return h
